# baseline (speedup 1.0000x reference)
import functools

import jax
import jax.numpy as jnp
from jax import lax
from jax.experimental import pallas as pl
from jax.experimental.pallas import tpu as pltpu
from jax.experimental.pallas import tpu_sc as plsc

NHEADS = 8
NDIM = 128
NCPH = NDIM // NHEADS
ROWW = 144
GROWW = 256
NWORK = 32
CHUNK = 128
NNODE = 10000
NEDGE = 20000
LANES = 16

_HIGH = lax.Precision.HIGHEST


def _expand_mat():
    h = lax.broadcasted_iota(jnp.int32, (NHEADS, NDIM), 0)
    d = lax.broadcasted_iota(jnp.int32, (NHEADS, NDIM), 1)
    return (d // NCPH == h).astype(jnp.float32)


def _layernorm(x, g, b, eps=1e-5):
    m = x.mean(-1, keepdims=True)
    v = ((x - m) ** 2).mean(-1, keepdims=True)
    return (x - m) / jnp.sqrt(v + eps) * g + b


BLK = 2000


def _amax_body(x_ref, wk_ref, bk_ref, att_ref, out_ref):
    x = x_ref[...]
    k = jnp.dot(x, wk_ref[...], precision=_HIGH) + bk_ref[...]
    r = _expand_mat()
    a = jnp.dot(k * att_ref[...], r.T, precision=_HIGH)
    bmax = jnp.max(a, axis=0, keepdims=True)

    @pl.when(pl.program_id(0) == 0)
    def _():
        out_ref[...] = jnp.full_like(out_ref, -jnp.inf)

    out_ref[...] = jnp.maximum(out_ref[...], bmax)


def _proj_body(x_ref, wk_ref, bk_ref, wv_ref, bv_ref, att_ref, gmax_ref,
               out_ref):
    x = x_ref[...]
    att = att_ref[...]
    k = jnp.dot(x, wk_ref[...], precision=_HIGH) + bk_ref[...]
    v = jnp.dot(x, wv_ref[...], precision=_HIGH) + bv_ref[...]
    r = _expand_mat()
    a = jnp.dot(k * att, r.T, precision=_HIGH)
    e = jnp.exp(a - gmax_ref[...])
    ex = jnp.dot(e, r, precision=_HIGH)
    u = v * ex
    pad = jnp.zeros((x.shape[0], GROWW - NDIM - NHEADS), jnp.float32)
    out_ref[...] = jnp.concatenate([u, e, pad], axis=1)


def _row_spec(width):
    return pl.BlockSpec((BLK, width), lambda i: (i, 0))


def _full_spec(shape):
    return pl.BlockSpec(shape, lambda i: tuple(0 for _ in shape))


def _proj(x, wk, bk, wv, bv, att_row):
    n = x.shape[0]
    grid = (n // BLK,)
    bk2 = bk.reshape(1, NDIM)
    gmax = pl.pallas_call(
        _amax_body,
        grid=grid,
        in_specs=[_row_spec(NDIM), _full_spec((NDIM, NDIM)),
                  _full_spec((1, NDIM)), _full_spec((1, NDIM))],
        out_specs=_full_spec((1, NHEADS)),
        out_shape=jax.ShapeDtypeStruct((1, NHEADS), jnp.float32),
    )(x, wk, bk2, att_row)
    return pl.pallas_call(
        _proj_body,
        grid=grid,
        in_specs=[_row_spec(NDIM), _full_spec((NDIM, NDIM)),
                  _full_spec((1, NDIM)), _full_spec((NDIM, NDIM)),
                  _full_spec((1, NDIM)), _full_spec((1, NDIM)),
                  _full_spec((1, NHEADS))],
        out_specs=_row_spec(GROWW),
        out_shape=jax.ShapeDtypeStruct((n, GROWW), jnp.float32),
    )(x, wk, bk2, wv, bv.reshape(1, NDIM), att_row, gmax)


def _segsum_sc(rows, map_pad, seg_pad, bounds, stripe):
    mesh = plsc.VectorSubcoreMesh(core_axis_name="c", subcore_axis_name="s",
                                  num_cores=2, num_subcores=16)
    nseg_pad = NWORK * stripe

    @functools.partial(
        pl.kernel,
        out_type=jax.ShapeDtypeStruct((nseg_pad * ROWW,), jnp.float32),
        mesh=mesh,
        scratch_types=[
            pltpu.VMEM((48,), jnp.int32),
            pltpu.VMEM((CHUNK,), jnp.int32),
            pltpu.VMEM((CHUNK,), jnp.int32),
            pltpu.VMEM((CHUNK,), jnp.int32),
            pltpu.VMEM((CHUNK,), jnp.int32),
            pltpu.VMEM((CHUNK, GROWW), jnp.float32),
            pltpu.VMEM((stripe * ROWW,), jnp.float32),
            pltpu.SemaphoreType.DMA,
            pltpu.SemaphoreType.DMA,
        ],
    )
    def k(rows_hbm, map_hbm, seg_hbm, bounds_hbm, out_hbm,
          bv, idx0, idx1, seg0v, seg1v, rows0, outv, sem0, sem1):
        idxs = (idx0, idx1)
        segs = (seg0v, seg1v)
        rows = (rows0, rows0)
        sems = (sem0, sem1)
        wid = lax.axis_index("s") * 2 + lax.axis_index("c")
        seg0 = wid * stripe
        nq = ROWW // LANES

        pltpu.sync_copy(bounds_hbm, bv)
        bvec = bv[pl.ds(wid, LANES)]
        lo = bvec[0]
        hi = bvec[1]
        lo_al = (lo // 8) * 8
        nch = (hi - lo_al + CHUNK - 1) // CHUNK

        zeros = jnp.zeros((LANES,), jnp.float32)

        def zero_row(r, _):
            for q in range(nq):
                outv[pl.ds(r * ROWW + q * LANES, LANES)] = zeros
            return 0

        lax.fori_loop(0, stripe, zero_row, 0)

        def start(t, ib):
            @pl.when(t < nch)
            def _():
                base = lo_al + t * CHUNK
                pltpu.sync_copy(map_hbm.at[pl.ds(base, CHUNK)], idxs[ib])
                pltpu.sync_copy(seg_hbm.at[pl.ds(base, CHUNK)], segs[ib])

        start(0, 0)
        start(1, 1)

        def process(t, ib):
            base = lo_al + t * CHUNK
            pltpu.async_copy(rows_hbm.at[idxs[ib]], rows[ib], sems[ib]).wait()

            def blk(bi, _):
                sv = segs[ib][pl.ds(bi * LANES, LANES)]
                for jj in range(LANES):
                    j = bi * LANES + jj
                    s = sv[jj]
                    g = base + j
                    ok = (g >= lo) & (g < hi)
                    off = (s - seg0) * ROWW

                    @pl.when(ok)
                    def _(off=off, j=j):
                        for q in range(nq):
                            plsc.addupdate(
                                outv.at[pl.ds(off + q * LANES, LANES)],
                                rows[ib][j, pl.ds(q * LANES, LANES)])
                return 0

            lax.fori_loop(0, CHUNK // LANES, blk, 0)
            start(t + 2, ib)

        def pair(tt, _):
            for ib in (0, 1):
                t = tt * 2 + ib

                @pl.when(t < nch)
                def _(t=t, ib=ib):
                    process(t, ib)
            return 0

        lax.fori_loop(0, (nch + 1) // 2, pair, 0)

        pltpu.sync_copy(outv, out_hbm.at[pl.ds(seg0 * ROWW, stripe * ROWW)])

    return k(rows, map_pad, seg_pad, bounds).reshape(nseg_pad, ROWW)


def _post_body(p_ref, att_ref, rw_ref, rb_ref, g0_ref, b0_ref,
               g1_ref, b1_ref, out_ref):
    p = p_ref[...]
    num = p[:, :NDIM]
    den = p[:, NDIM:NDIM + NHEADS]
    r = _expand_mat()
    denx = jnp.dot(den, r, precision=_HIGH)
    out = num / (denx + 1e-30) + att_ref[...]
    out = _layernorm(out, g0_ref[...], b0_ref[...])
    h = jnp.maximum(jnp.dot(out, rw_ref[...], precision=_HIGH) + rb_ref[...], 0.0)
    out = _layernorm(out + h, g1_ref[...], b1_ref[...])
    out_ref[...] = jnp.maximum(out, 0.0)


def _post(p, att_row, rw, rb, g0, b0, g1, b1):
    n = p.shape[0]
    return pl.pallas_call(
        _post_body,
        grid=(n // BLK,),
        in_specs=[_row_spec(ROWW), _full_spec((1, NDIM)),
                  _full_spec((NDIM, NDIM)), _full_spec((1, NDIM)),
                  _full_spec((1, NDIM)), _full_spec((1, NDIM)),
                  _full_spec((1, NDIM)), _full_spec((1, NDIM))],
        out_specs=_row_spec(NDIM),
        out_shape=jax.ShapeDtypeStruct((n, NDIM), jnp.float32),
    )(p, att_row, rw, rb.reshape(1, NDIM), g0.reshape(1, NDIM),
      b0.reshape(1, NDIM), g1.reshape(1, NDIM), b1.reshape(1, NDIM))


def _cls_body(x_ref, w_ref, b_ref, out_ref):
    z = jnp.dot(x_ref[...], w_ref[...], precision=_HIGH) + b_ref[...]
    m = jnp.max(z, axis=-1, keepdims=True)
    lse = m + jnp.log(jnp.sum(jnp.exp(z - m), axis=-1, keepdims=True))
    out_ref[...] = z - lse


def _classifier(x, w, b):
    n, ncls = x.shape[0], w.shape[1]
    return pl.pallas_call(
        _cls_body,
        grid=(n // BLK,),
        in_specs=[_row_spec(NDIM), _full_spec((NDIM, ncls)),
                  _full_spec((1, ncls))],
        out_specs=_row_spec(ncls),
        out_shape=jax.ShapeDtypeStruct((n, ncls), jnp.float32),
    )(x, w, b.reshape(1, ncls))


def _stripe_setup(seg, nseg):
    stripe = -(-nseg // NWORK)
    edges = jnp.arange(NWORK + 1, dtype=jnp.int32) * stripe
    bounds = jnp.searchsorted(seg, edges, side="left").astype(jnp.int32)
    bounds = jnp.concatenate(
        [bounds, jnp.zeros((48 - NWORK - 1,), jnp.int32)])
    seg_pad = jnp.concatenate(
        [seg, jnp.full((CHUNK,), NWORK * stripe, jnp.int32)])
    return stripe, bounds, seg_pad


def _pma(x, mp, sg, bounds, stripe, nseg,
         wk, bk, wv, bv, att, rw, rb, g0, b0, g1, b1):
    att_row = att.reshape(1, NDIM)
    w = _proj(x, wk, bk, wv, bv, att_row)
    p = _segsum_sc(w, mp, sg, bounds, stripe)
    return _post(p[:nseg], att_row, rw, rb, g0, b0, g1, b1)


def kernel(node_x, nodes_map, eb_batch, edges_map, nb_batch,
           n2e_WK, n2e_bK, n2e_WV, n2e_bV, n2e_att, n2e_rW, n2e_rb,
           n2e_g0, n2e_b0, n2e_g1, n2e_b1,
           e2n_WK, e2n_bK, e2n_WV, e2n_bV, e2n_att, e2n_rW, e2n_rb,
           e2n_g0, e2n_b0, e2n_g1, e2n_b1,
           cls_W, cls_b):
    pad0 = jnp.zeros((CHUNK,), jnp.int32)
    mp1 = jnp.concatenate([nodes_map, pad0])
    mp2 = jnp.concatenate([edges_map, pad0])
    st1, bd1, sg1 = _stripe_setup(eb_batch, NEDGE)
    st2, bd2, sg2 = _stripe_setup(nb_batch, NNODE)

    x = node_x
    for i in range(2):
        ex = _pma(x, mp1, sg1, bd1, st1, NEDGE,
                  n2e_WK[i], n2e_bK[i], n2e_WV[i], n2e_bV[i], n2e_att[i],
                  n2e_rW[i], n2e_rb[i], n2e_g0[i], n2e_b0[i],
                  n2e_g1[i], n2e_b1[i])
        x = _pma(ex, mp2, sg2, bd2, st2, NNODE,
                 e2n_WK[i], e2n_bK[i], e2n_WV[i], e2n_bV[i], e2n_att[i],
                 e2n_rW[i], e2n_rb[i], e2n_g0[i], e2n_b0[i],
                 e2n_g1[i], e2n_b1[i])
    return _classifier(x, cls_W, cls_b)


# speedup vs baseline: 1.6591x; 1.6591x over previous
import functools

import jax
import jax.numpy as jnp
from jax import lax
from jax.experimental import pallas as pl
from jax.experimental.pallas import tpu as pltpu
from jax.experimental.pallas import tpu_sc as plsc

NHEADS = 8
NDIM = 128
NCPH = NDIM // NHEADS
ROWW = 144
GROWW = 256
NWORK = 32
CHUNK = 64
NNODE = 10000
NEDGE = 20000
LANES = 16

_HIGH = lax.Precision.HIGHEST


def _expand_mat():
    h = lax.broadcasted_iota(jnp.int32, (NHEADS, NDIM), 0)
    d = lax.broadcasted_iota(jnp.int32, (NHEADS, NDIM), 1)
    return (d // NCPH == h).astype(jnp.float32)


def _layernorm(x, g, b, eps=1e-5):
    m = x.mean(-1, keepdims=True)
    v = ((x - m) ** 2).mean(-1, keepdims=True)
    return (x - m) / jnp.sqrt(v + eps) * g + b


BLK = 2000


def _amax_body(x_ref, wk_ref, bk_ref, att_ref, out_ref):
    x = x_ref[...]
    k = jnp.dot(x, wk_ref[...], precision=_HIGH) + bk_ref[...]
    r = _expand_mat()
    a = jnp.dot(k * att_ref[...], r.T, precision=_HIGH)
    bmax = jnp.max(a, axis=0, keepdims=True)

    @pl.when(pl.program_id(0) == 0)
    def _():
        out_ref[...] = jnp.full_like(out_ref, -jnp.inf)

    out_ref[...] = jnp.maximum(out_ref[...], bmax)


def _proj_body(x_ref, wk_ref, bk_ref, wv_ref, bv_ref, att_ref, gmax_ref,
               out_ref):
    x = x_ref[...]
    att = att_ref[...]
    k = jnp.dot(x, wk_ref[...], precision=_HIGH) + bk_ref[...]
    v = jnp.dot(x, wv_ref[...], precision=_HIGH) + bv_ref[...]
    r = _expand_mat()
    a = jnp.dot(k * att, r.T, precision=_HIGH)
    e = jnp.exp(a - gmax_ref[...])
    ex = jnp.dot(e, r, precision=_HIGH)
    u = v * ex
    pad = jnp.zeros((x.shape[0], GROWW - NDIM - NHEADS), jnp.float32)
    out_ref[...] = jnp.concatenate([u, e, pad], axis=1)


def _row_spec(width):
    return pl.BlockSpec((BLK, width), lambda i: (i, 0))


def _full_spec(shape):
    return pl.BlockSpec(shape, lambda i: tuple(0 for _ in shape))


def _proj(x, wk, bk, wv, bv, att_row):
    n = x.shape[0]
    grid = (n // BLK,)
    bk2 = bk.reshape(1, NDIM)
    gmax = pl.pallas_call(
        _amax_body,
        grid=grid,
        in_specs=[_row_spec(NDIM), _full_spec((NDIM, NDIM)),
                  _full_spec((1, NDIM)), _full_spec((1, NDIM))],
        out_specs=_full_spec((1, NHEADS)),
        out_shape=jax.ShapeDtypeStruct((1, NHEADS), jnp.float32),
    )(x, wk, bk2, att_row)
    return pl.pallas_call(
        _proj_body,
        grid=grid,
        in_specs=[_row_spec(NDIM), _full_spec((NDIM, NDIM)),
                  _full_spec((1, NDIM)), _full_spec((NDIM, NDIM)),
                  _full_spec((1, NDIM)), _full_spec((1, NDIM)),
                  _full_spec((1, NHEADS))],
        out_specs=_row_spec(GROWW),
        out_shape=jax.ShapeDtypeStruct((n, GROWW), jnp.float32),
    )(x, wk, bk2, wv, bv.reshape(1, NDIM), att_row, gmax)


def _segsum_sc(rows, map_pad, seg_pad, bounds, stripe):
    mesh = plsc.VectorSubcoreMesh(core_axis_name="c", subcore_axis_name="s",
                                  num_cores=2, num_subcores=16)
    nseg_pad = NWORK * stripe

    @functools.partial(
        pl.kernel,
        out_type=jax.ShapeDtypeStruct((nseg_pad * ROWW,), jnp.float32),
        mesh=mesh,
        scratch_types=[
            pltpu.VMEM((48,), jnp.int32),
            pltpu.VMEM((CHUNK,), jnp.int32),
            pltpu.VMEM((CHUNK,), jnp.int32),
            pltpu.VMEM((CHUNK,), jnp.int32),
            pltpu.VMEM((CHUNK,), jnp.int32),
            pltpu.VMEM((CHUNK, GROWW), jnp.float32),
            pltpu.VMEM((CHUNK, GROWW), jnp.float32),
            pltpu.VMEM((stripe * ROWW,), jnp.float32),
            pltpu.SemaphoreType.DMA,
            pltpu.SemaphoreType.DMA,
        ],
    )
    def k(rows_hbm, map_hbm, seg_hbm, bounds_hbm, out_hbm,
          bv, idx0, idx1, seg0v, seg1v, rows0, rows1, outv, sem0, sem1):
        idxs = (idx0, idx1)
        segs = (seg0v, seg1v)
        rows = (rows0, rows1)
        sems = (sem0, sem1)
        wid = lax.axis_index("s") * 2 + lax.axis_index("c")
        seg0 = wid * stripe
        nq = ROWW // LANES

        pltpu.sync_copy(bounds_hbm, bv)
        bvec = bv[pl.ds(wid, LANES)]
        lo = bvec[0]
        hi = bvec[1]
        lo_al = (lo // 8) * 8
        nch = (hi - lo_al + CHUNK - 1) // CHUNK
        nch2 = 2 * ((nch + 1) // 2)

        zeros = jnp.zeros((LANES,), jnp.float32)

        def zero_row(r, _):
            for q in range(nq):
                outv[pl.ds(r * ROWW + q * LANES, LANES)] = zeros
            return 0

        lax.fori_loop(0, stripe, zero_row, 0)

        def start(t, ib):
            @pl.when(t < nch2)
            def _():
                base = lo_al + t * CHUNK
                pltpu.sync_copy(map_hbm.at[pl.ds(base, CHUNK)], idxs[ib])
                pltpu.sync_copy(seg_hbm.at[pl.ds(base, CHUNK)], segs[ib])
                pltpu.async_copy(rows_hbm.at[idxs[ib]], rows[ib], sems[ib])

        start(0, 0)
        start(1, 1)

        def process(t, ib, carry):
            base = lo_al + t * CHUNK
            pltpu.make_async_copy(
                rows_hbm.at[idxs[ib]], rows[ib], sems[ib]).wait()

            def blk(bi, c):
                cur = c[0]
                acc = list(c[1:])
                sv = segs[ib][pl.ds(bi * LANES, LANES)]
                for jj in range(LANES):
                    j = bi * LANES + jj
                    s = sv[jj]
                    g = base + j
                    ok = (g >= lo) & (g < hi)
                    same = ok & (s == cur)
                    flush = ok & jnp.logical_not(s == cur) & (cur >= 0)
                    accs = tuple(acc)

                    @pl.when(flush)
                    def _(cur=cur, accs=accs):
                        r = cur - seg0
                        for q in range(nq):
                            outv[pl.ds(r * ROWW + q * LANES, LANES)] = accs[q]

                    for q in range(nq):
                        row = rows[ib][j, pl.ds(q * LANES, LANES)]
                        acc[q] = jnp.where(
                            ok, jnp.where(same, acc[q] + row, row), acc[q])
                    cur = jnp.where(ok, s, cur)
                return (cur, *acc)

            carry = lax.fori_loop(0, CHUNK // LANES, blk, carry)
            start(t + 2, ib)
            return carry

        def pair(tt, carry):
            carry = process(tt * 2, 0, carry)
            carry = process(tt * 2 + 1, 1, carry)
            return carry

        init = (jnp.int32(-1),) + tuple(
            jnp.zeros((LANES,), jnp.float32) for _ in range(nq))
        fin = lax.fori_loop(0, nch2 // 2, pair, init)
        cur = fin[0]
        acc = fin[1:]

        @pl.when(cur >= 0)
        def _():
            r = cur - seg0
            for q in range(nq):
                outv[pl.ds(r * ROWW + q * LANES, LANES)] = acc[q]

        pltpu.sync_copy(outv, out_hbm.at[pl.ds(seg0 * ROWW, stripe * ROWW)])

    return k(rows, map_pad, seg_pad, bounds).reshape(nseg_pad, ROWW)


def _post_body(p_ref, att_ref, rw_ref, rb_ref, g0_ref, b0_ref,
               g1_ref, b1_ref, out_ref):
    p = p_ref[...]
    num = p[:, :NDIM]
    den = p[:, NDIM:NDIM + NHEADS]
    r = _expand_mat()
    denx = jnp.dot(den, r, precision=_HIGH)
    out = num / (denx + 1e-30) + att_ref[...]
    out = _layernorm(out, g0_ref[...], b0_ref[...])
    h = jnp.maximum(jnp.dot(out, rw_ref[...], precision=_HIGH) + rb_ref[...], 0.0)
    out = _layernorm(out + h, g1_ref[...], b1_ref[...])
    out_ref[...] = jnp.maximum(out, 0.0)


def _post(p, att_row, rw, rb, g0, b0, g1, b1):
    n = p.shape[0]
    return pl.pallas_call(
        _post_body,
        grid=(n // BLK,),
        in_specs=[_row_spec(ROWW), _full_spec((1, NDIM)),
                  _full_spec((NDIM, NDIM)), _full_spec((1, NDIM)),
                  _full_spec((1, NDIM)), _full_spec((1, NDIM)),
                  _full_spec((1, NDIM)), _full_spec((1, NDIM))],
        out_specs=_row_spec(NDIM),
        out_shape=jax.ShapeDtypeStruct((n, NDIM), jnp.float32),
    )(p, att_row, rw, rb.reshape(1, NDIM), g0.reshape(1, NDIM),
      b0.reshape(1, NDIM), g1.reshape(1, NDIM), b1.reshape(1, NDIM))


def _cls_body(x_ref, w_ref, b_ref, out_ref):
    z = jnp.dot(x_ref[...], w_ref[...], precision=_HIGH) + b_ref[...]
    m = jnp.max(z, axis=-1, keepdims=True)
    lse = m + jnp.log(jnp.sum(jnp.exp(z - m), axis=-1, keepdims=True))
    out_ref[...] = z - lse


def _classifier(x, w, b):
    n, ncls = x.shape[0], w.shape[1]
    return pl.pallas_call(
        _cls_body,
        grid=(n // BLK,),
        in_specs=[_row_spec(NDIM), _full_spec((NDIM, ncls)),
                  _full_spec((1, ncls))],
        out_specs=_row_spec(ncls),
        out_shape=jax.ShapeDtypeStruct((n, ncls), jnp.float32),
    )(x, w, b.reshape(1, ncls))


def _stripe_setup(seg, nseg):
    stripe = -(-nseg // NWORK)
    edges = jnp.arange(NWORK + 1, dtype=jnp.int32) * stripe
    bounds = jnp.searchsorted(seg, edges, side="left").astype(jnp.int32)
    bounds = jnp.concatenate(
        [bounds, jnp.zeros((48 - NWORK - 1,), jnp.int32)])
    seg_pad = jnp.concatenate(
        [seg, jnp.full((2 * CHUNK,), NWORK * stripe, jnp.int32)])
    return stripe, bounds, seg_pad


def _pma(x, mp, sg, bounds, stripe, nseg,
         wk, bk, wv, bv, att, rw, rb, g0, b0, g1, b1):
    att_row = att.reshape(1, NDIM)
    w = _proj(x, wk, bk, wv, bv, att_row)
    p = _segsum_sc(w, mp, sg, bounds, stripe)
    return _post(p[:nseg], att_row, rw, rb, g0, b0, g1, b1)


def kernel(node_x, nodes_map, eb_batch, edges_map, nb_batch,
           n2e_WK, n2e_bK, n2e_WV, n2e_bV, n2e_att, n2e_rW, n2e_rb,
           n2e_g0, n2e_b0, n2e_g1, n2e_b1,
           e2n_WK, e2n_bK, e2n_WV, e2n_bV, e2n_att, e2n_rW, e2n_rb,
           e2n_g0, e2n_b0, e2n_g1, e2n_b1,
           cls_W, cls_b):
    pad0 = jnp.zeros((2 * CHUNK,), jnp.int32)
    mp1 = jnp.concatenate([nodes_map, pad0])
    mp2 = jnp.concatenate([edges_map, pad0])
    st1, bd1, sg1 = _stripe_setup(eb_batch, NEDGE)
    st2, bd2, sg2 = _stripe_setup(nb_batch, NNODE)

    x = node_x
    for i in range(2):
        ex = _pma(x, mp1, sg1, bd1, st1, NEDGE,
                  n2e_WK[i], n2e_bK[i], n2e_WV[i], n2e_bV[i], n2e_att[i],
                  n2e_rW[i], n2e_rb[i], n2e_g0[i], n2e_b0[i],
                  n2e_g1[i], n2e_b1[i])
        x = _pma(ex, mp2, sg2, bd2, st2, NNODE,
                 e2n_WK[i], e2n_bK[i], e2n_WV[i], e2n_bV[i], e2n_att[i],
                 e2n_rW[i], e2n_rb[i], e2n_g0[i], e2n_b0[i],
                 e2n_g1[i], e2n_b1[i])
    return _classifier(x, cls_W, cls_b)


# speedup vs baseline: 1.8883x; 1.1381x over previous
import functools

import jax
import jax.numpy as jnp
from jax import lax
from jax.experimental import pallas as pl
from jax.experimental.pallas import tpu as pltpu
from jax.experimental.pallas import tpu_sc as plsc

NHEADS = 8
NDIM = 128
NCPH = NDIM // NHEADS
ROWW = 144
GROWW = 128
NWORK = 32
CHUNK = 128
NNODE = 10000
NEDGE = 20000
LANES = 16

_HIGH = lax.Precision.HIGHEST


def _expand_mat():
    h = lax.broadcasted_iota(jnp.int32, (NHEADS, NDIM), 0)
    d = lax.broadcasted_iota(jnp.int32, (NHEADS, NDIM), 1)
    return (d // NCPH == h).astype(jnp.float32)


def _layernorm(x, g, b, eps=1e-5):
    m = x.mean(-1, keepdims=True)
    v = ((x - m) ** 2).mean(-1, keepdims=True)
    return (x - m) / jnp.sqrt(v + eps) * g + b


BLK = 2000


def _amax_body(x_ref, wk_ref, bk_ref, att_ref, out_ref):
    x = x_ref[...]
    k = jnp.dot(x, wk_ref[...], precision=_HIGH) + bk_ref[...]
    r = _expand_mat()
    a = jnp.dot(k * att_ref[...], r.T, precision=_HIGH)
    bmax = jnp.max(a, axis=0, keepdims=True)

    @pl.when(pl.program_id(0) == 0)
    def _():
        out_ref[...] = jnp.full_like(out_ref, -jnp.inf)

    out_ref[...] = jnp.maximum(out_ref[...], bmax)


def _proj_body(x_ref, wk_ref, bk_ref, wv_ref, bv_ref, att_ref, gmax_ref,
               out_ref):
    x = x_ref[...]
    att = att_ref[...]
    k = jnp.dot(x, wk_ref[...], precision=_HIGH) + bk_ref[...]
    v = jnp.dot(x, wv_ref[...], precision=_HIGH) + bv_ref[...]
    r = _expand_mat()
    a = jnp.dot(k * att, r.T, precision=_HIGH)
    e = jnp.exp(a - gmax_ref[...])
    ex = jnp.dot(e, r, precision=_HIGH)
    u = v * ex
    cw = lax.broadcasted_iota(jnp.int32, (NDIM, NDIM // 2), 0)
    ww = lax.broadcasted_iota(jnp.int32, (NDIM, NDIM // 2), 1)
    lo_src = 32 * (ww // LANES) + ww % LANES
    plo = (cw == lo_src).astype(jnp.float32)
    phi = (cw == lo_src + LANES).astype(jnp.float32)
    lo16 = lax.bitcast_convert_type(
        jnp.dot(u, plo, precision=_HIGH).astype(jnp.bfloat16), jnp.uint16)
    hi16 = lax.bitcast_convert_type(
        jnp.dot(u, phi, precision=_HIGH).astype(jnp.bfloat16), jnp.uint16)
    w32 = lo16.astype(jnp.uint32) | (hi16.astype(jnp.uint32) << 16)
    packed = lax.bitcast_convert_type(w32, jnp.float32)
    pad = jnp.zeros(
        (x.shape[0], GROWW - NDIM // 2 - NHEADS), jnp.float32)
    out_ref[...] = jnp.concatenate([packed, e, pad], axis=1)


def _row_spec(width):
    return pl.BlockSpec((BLK, width), lambda i: (i, 0))


def _full_spec(shape):
    return pl.BlockSpec(shape, lambda i: tuple(0 for _ in shape))


def _proj(x, wk, bk, wv, bv, att_row):
    n = x.shape[0]
    grid = (n // BLK,)
    bk2 = bk.reshape(1, NDIM)
    gmax = pl.pallas_call(
        _amax_body,
        grid=grid,
        in_specs=[_row_spec(NDIM), _full_spec((NDIM, NDIM)),
                  _full_spec((1, NDIM)), _full_spec((1, NDIM))],
        out_specs=_full_spec((1, NHEADS)),
        out_shape=jax.ShapeDtypeStruct((1, NHEADS), jnp.float32),
    )(x, wk, bk2, att_row)
    return pl.pallas_call(
        _proj_body,
        grid=grid,
        in_specs=[_row_spec(NDIM), _full_spec((NDIM, NDIM)),
                  _full_spec((1, NDIM)), _full_spec((NDIM, NDIM)),
                  _full_spec((1, NDIM)), _full_spec((1, NDIM)),
                  _full_spec((1, NHEADS))],
        out_specs=_row_spec(GROWW),
        out_shape=jax.ShapeDtypeStruct((n, GROWW), jnp.float32),
    )(x, wk, bk2, wv, bv.reshape(1, NDIM), att_row, gmax)


def _segsum_sc(rows, map_pad, seg_pad, bounds, stripe):
    mesh = plsc.VectorSubcoreMesh(core_axis_name="c", subcore_axis_name="s",
                                  num_cores=2, num_subcores=16)
    nseg_pad = NWORK * stripe

    @functools.partial(
        pl.kernel,
        out_type=jax.ShapeDtypeStruct((nseg_pad * ROWW,), jnp.float32),
        mesh=mesh,
        scratch_types=[
            pltpu.VMEM((48,), jnp.int32),
            pltpu.VMEM((CHUNK,), jnp.int32),
            pltpu.VMEM((CHUNK,), jnp.int32),
            pltpu.VMEM((CHUNK,), jnp.int32),
            pltpu.VMEM((CHUNK,), jnp.int32),
            pltpu.VMEM((CHUNK, GROWW), jnp.float32),
            pltpu.VMEM((CHUNK, GROWW), jnp.float32),
            pltpu.VMEM((stripe * ROWW,), jnp.float32),
            pltpu.SemaphoreType.DMA,
            pltpu.SemaphoreType.DMA,
        ],
        compiler_params=pltpu.CompilerParams(needs_layout_passes=False),
    )
    def k(rows_hbm, map_hbm, seg_hbm, bounds_hbm, out_hbm,
          bv, idx0, idx1, seg0v, seg1v, rows0, rows1, outv, sem0, sem1):
        idxs = (idx0, idx1)
        segs = (seg0v, seg1v)
        rows = (rows0, rows1)
        sems = (sem0, sem1)
        wid = lax.axis_index("s") * 2 + lax.axis_index("c")
        seg0 = wid * stripe
        nq = ROWW // LANES

        pltpu.sync_copy(bounds_hbm, bv)
        bvec = bv[pl.ds(wid, LANES)]
        lo = bvec[0]
        hi = bvec[1]
        lo_al = (lo // 8) * 8
        nch = (hi - lo_al + CHUNK - 1) // CHUNK
        nch2 = 2 * ((nch + 1) // 2)

        zeros = jnp.zeros((LANES,), jnp.float32)

        def zero_row(r, _):
            for q in range(nq):
                outv[pl.ds(r * ROWW + q * LANES, LANES)] = zeros
            return 0

        lax.fori_loop(0, stripe, zero_row, 0)

        def start(t, ib):
            @pl.when(t < nch2)
            def _():
                base = lo_al + t * CHUNK
                pltpu.sync_copy(map_hbm.at[pl.ds(base, CHUNK)], idxs[ib])
                pltpu.sync_copy(seg_hbm.at[pl.ds(base, CHUNK)], segs[ib])
                pltpu.async_copy(rows_hbm.at[idxs[ib]], rows[ib], sems[ib])

        start(0, 0)
        start(1, 1)

        def process(t, ib, carry):
            base = lo_al + t * CHUNK
            pltpu.make_async_copy(
                rows_hbm.at[idxs[ib]], rows[ib], sems[ib]).wait()

            def blk(bi, c):
                cur = c[0]
                acc = list(c[1:])
                sv = segs[ib][pl.ds(bi * LANES, LANES)]
                for jj in range(LANES):
                    j = bi * LANES + jj
                    s = sv[jj]
                    g = base + j
                    ok = (g >= lo) & (g < hi)
                    same = ok & (s == cur)
                    flush = ok & jnp.logical_not(s == cur) & (cur >= 0)
                    accs = tuple(acc)

                    @pl.when(flush)
                    def _(cur=cur, accs=accs):
                        r = cur - seg0
                        for q in range(nq):
                            outv[pl.ds(r * ROWW + q * LANES, LANES)] = accs[q]

                    rowv = []
                    for q in range(4):
                        pu = rows[ib][j, pl.ds(q * LANES, LANES)]
                        pb = plsc.bitcast(pu, jnp.bfloat16)
                        ra, rb = plsc.unpack(
                            pb, format=plsc.PackFormat.INTERLEAVED)
                        rowv.append(ra)
                        rowv.append(rb)
                    rowv.append(rows[ib][j, pl.ds(4 * LANES, LANES)])
                    for q in range(nq):
                        acc[q] = jnp.where(
                            ok, jnp.where(same, acc[q] + rowv[q], rowv[q]),
                            acc[q])
                    cur = jnp.where(ok, s, cur)
                return (cur, *acc)

            carry = lax.fori_loop(0, CHUNK // LANES, blk, carry)
            start(t + 2, ib)
            return carry

        def pair(tt, carry):
            carry = process(tt * 2, 0, carry)
            carry = process(tt * 2 + 1, 1, carry)
            return carry

        init = (jnp.int32(-1),) + tuple(
            jnp.zeros((LANES,), jnp.float32) for _ in range(nq))
        fin = lax.fori_loop(0, nch2 // 2, pair, init)
        cur = fin[0]
        acc = fin[1:]

        @pl.when(cur >= 0)
        def _():
            r = cur - seg0
            for q in range(nq):
                outv[pl.ds(r * ROWW + q * LANES, LANES)] = acc[q]

        pltpu.sync_copy(outv, out_hbm.at[pl.ds(seg0 * ROWW, stripe * ROWW)])

    return k(rows, map_pad, seg_pad, bounds).reshape(nseg_pad, ROWW)


def _post_body(p_ref, att_ref, rw_ref, rb_ref, g0_ref, b0_ref,
               g1_ref, b1_ref, out_ref):
    p = p_ref[...]
    num = p[:, :NDIM]
    den = p[:, NDIM:NDIM + NHEADS]
    r = _expand_mat()
    denx = jnp.dot(den, r, precision=_HIGH)
    out = num / (denx + 1e-30) + att_ref[...]
    out = _layernorm(out, g0_ref[...], b0_ref[...])
    h = jnp.maximum(jnp.dot(out, rw_ref[...], precision=_HIGH) + rb_ref[...], 0.0)
    out = _layernorm(out + h, g1_ref[...], b1_ref[...])
    out_ref[...] = jnp.maximum(out, 0.0)


def _post(p, att_row, rw, rb, g0, b0, g1, b1):
    n = p.shape[0]
    return pl.pallas_call(
        _post_body,
        grid=(n // BLK,),
        in_specs=[_row_spec(ROWW), _full_spec((1, NDIM)),
                  _full_spec((NDIM, NDIM)), _full_spec((1, NDIM)),
                  _full_spec((1, NDIM)), _full_spec((1, NDIM)),
                  _full_spec((1, NDIM)), _full_spec((1, NDIM))],
        out_specs=_row_spec(NDIM),
        out_shape=jax.ShapeDtypeStruct((n, NDIM), jnp.float32),
    )(p, att_row, rw, rb.reshape(1, NDIM), g0.reshape(1, NDIM),
      b0.reshape(1, NDIM), g1.reshape(1, NDIM), b1.reshape(1, NDIM))


def _cls_body(x_ref, w_ref, b_ref, out_ref):
    z = jnp.dot(x_ref[...], w_ref[...], precision=_HIGH) + b_ref[...]
    m = jnp.max(z, axis=-1, keepdims=True)
    lse = m + jnp.log(jnp.sum(jnp.exp(z - m), axis=-1, keepdims=True))
    out_ref[...] = z - lse


def _classifier(x, w, b):
    n, ncls = x.shape[0], w.shape[1]
    return pl.pallas_call(
        _cls_body,
        grid=(n // BLK,),
        in_specs=[_row_spec(NDIM), _full_spec((NDIM, ncls)),
                  _full_spec((1, ncls))],
        out_specs=_row_spec(ncls),
        out_shape=jax.ShapeDtypeStruct((n, ncls), jnp.float32),
    )(x, w, b.reshape(1, ncls))


def _stripe_setup(seg, nseg):
    stripe = -(-nseg // NWORK)
    edges = jnp.arange(NWORK + 1, dtype=jnp.int32) * stripe
    bounds = jnp.searchsorted(seg, edges, side="left").astype(jnp.int32)
    bounds = jnp.concatenate(
        [bounds, jnp.zeros((48 - NWORK - 1,), jnp.int32)])
    seg_pad = jnp.concatenate(
        [seg, jnp.full((2 * CHUNK,), NWORK * stripe, jnp.int32)])
    return stripe, bounds, seg_pad


def _pma(x, mp, sg, bounds, stripe, nseg,
         wk, bk, wv, bv, att, rw, rb, g0, b0, g1, b1):
    att_row = att.reshape(1, NDIM)
    w = _proj(x, wk, bk, wv, bv, att_row)
    p = _segsum_sc(w, mp, sg, bounds, stripe)
    return _post(p[:nseg], att_row, rw, rb, g0, b0, g1, b1)


def kernel(node_x, nodes_map, eb_batch, edges_map, nb_batch,
           n2e_WK, n2e_bK, n2e_WV, n2e_bV, n2e_att, n2e_rW, n2e_rb,
           n2e_g0, n2e_b0, n2e_g1, n2e_b1,
           e2n_WK, e2n_bK, e2n_WV, e2n_bV, e2n_att, e2n_rW, e2n_rb,
           e2n_g0, e2n_b0, e2n_g1, e2n_b1,
           cls_W, cls_b):
    pad0 = jnp.zeros((2 * CHUNK,), jnp.int32)
    mp1 = jnp.concatenate([nodes_map, pad0])
    mp2 = jnp.concatenate([edges_map, pad0])
    st1, bd1, sg1 = _stripe_setup(eb_batch, NEDGE)
    st2, bd2, sg2 = _stripe_setup(nb_batch, NNODE)

    x = node_x
    for i in range(2):
        ex = _pma(x, mp1, sg1, bd1, st1, NEDGE,
                  n2e_WK[i], n2e_bK[i], n2e_WV[i], n2e_bV[i], n2e_att[i],
                  n2e_rW[i], n2e_rb[i], n2e_g0[i], n2e_b0[i],
                  n2e_g1[i], n2e_b1[i])
        x = _pma(ex, mp2, sg2, bd2, st2, NNODE,
                 e2n_WK[i], e2n_bK[i], e2n_WV[i], e2n_bV[i], e2n_att[i],
                 e2n_rW[i], e2n_rb[i], e2n_g0[i], e2n_b0[i],
                 e2n_g1[i], e2n_b1[i])
    return _classifier(x, cls_W, cls_b)
